# baseline (speedup 1.0000x reference)
import functools

import jax
import jax.numpy as jnp
import numpy as np
from jax import lax
from jax.experimental import pallas as pl
from jax.experimental.pallas import tpu as pltpu
from jax.experimental.pallas import tpu_sc as plsc

N = 32768
D = 768
H = 64
O = 16
E = 8
PER = N // E

TILE = 2048
SC_NS = 16
SC_CHUNK = N // SC_NS
SC_K = SC_CHUNK // 128


def _sc_route_eid(idx3, eid3):
    mesh = plsc.VectorSubcoreMesh(
        core_axis_name="core", subcore_axis_name="subcore", num_cores=1)

    @functools.partial(
        pl.kernel,
        out_type=jax.ShapeDtypeStruct((N,), jnp.int32),
        mesh=mesh,
        scratch_types=[
            pltpu.VMEM((SC_K, 128), jnp.int32),
            pltpu.VMEM((SC_K, 128), jnp.int32),
            pltpu.VMEM((SC_CHUNK,), jnp.int32),
            pltpu.VMEM_SHARED((N,), jnp.int32),
            pltpu.SemaphoreType.DMA,
        ],
    )
    def sc_kernel(idx_hbm, eid_hbm, out_hbm, idx_v, val_v, stage_v, shared, sem):
        sid = lax.axis_index("subcore")
        base = sid * SC_CHUNK
        c_idx = pltpu.async_copy(idx_hbm.at[sid], idx_v, sem)
        c_val = pltpu.async_copy(eid_hbm.at[sid], val_v, sem)
        c_idx.wait()
        c_val.wait()
        scatters = [
            pltpu.async_copy(val_v.at[j], shared.at[idx_v.at[j]], sem)
            for j in range(SC_K)
        ]
        for c in scatters:
            c.wait()
        plsc.subcore_barrier()
        pltpu.sync_copy(shared.at[pl.ds(base, SC_CHUNK)], stage_v)
        pltpu.sync_copy(stage_v, out_hbm.at[pl.ds(base, SC_CHUNK)])

    return sc_kernel(idx3, eid3)


def _mlp_kernel(emb_ref, eid_ref, w1_ref, b1_ref, w2_ref, out_ref):
    x = emb_ref[...].astype(jnp.bfloat16)
    h = jax.lax.dot_general(
        w1_ref[...], x, (((0,), (1,)), ((), ())),
        preferred_element_type=jnp.float32,
    ) + b1_ref[...]
    h = h * jax.nn.sigmoid(h)
    eid = eid_ref[...]
    slot = jax.lax.broadcasted_iota(jnp.int32, (E * H, 1), 0) // H
    hm = jnp.where(eid == slot, h, 0.0).astype(jnp.bfloat16)
    oh8 = jnp.where(
        eid == jax.lax.broadcasted_iota(jnp.int32, (E, 1), 0),
        jnp.float32(1.0), jnp.float32(0.0)).astype(jnp.bfloat16)
    hm_aug = jnp.concatenate([hm, oh8], axis=0)
    out_ref[...] = jax.lax.dot_general(
        hm_aug, w2_ref[...], (((0,), (0,)), ((), ())),
        preferred_element_type=jnp.float32,
    )


def kernel(species, embedding, idx_0, idx_1, idx_2, idx_3, idx_4, idx_5,
           idx_6, idx_7, W1, b1, W2, b2):
    idx_cat = jnp.concatenate(
        [idx_0, idx_1, idx_2, idx_3, idx_4, idx_5, idx_6, idx_7])
    idx3 = idx_cat.astype(jnp.int32).reshape(SC_NS, SC_K, 128)
    eid3 = jnp.broadcast_to(
        jnp.arange(E, dtype=jnp.int32)[:, None], (E, PER)
    ).reshape(SC_NS, SC_K, 128)

    eid = _sc_route_eid(idx3, eid3).reshape(1, N)

    w1_cat = jnp.transpose(W1, (1, 0, 2)).reshape(D, E * H).astype(jnp.bfloat16)
    b1_cat = b1.reshape(E * H, 1)
    w2_aug = jnp.concatenate(
        [W2.reshape(E * H, O), b2], axis=0).astype(jnp.bfloat16)

    out = pl.pallas_call(
        _mlp_kernel,
        grid=(N // TILE,),
        in_specs=[
            pl.BlockSpec((TILE, D), lambda i: (i, 0)),
            pl.BlockSpec((1, TILE), lambda i: (0, i)),
            pl.BlockSpec((D, E * H), lambda i: (0, 0)),
            pl.BlockSpec((E * H, 1), lambda i: (0, 0)),
            pl.BlockSpec((E * H + E, O), lambda i: (0, 0)),
        ],
        out_specs=pl.BlockSpec((TILE, O), lambda i: (i, 0)),
        out_shape=jax.ShapeDtypeStruct((N, O), jnp.float32),
        compiler_params=pltpu.CompilerParams(
            dimension_semantics=("arbitrary",),
        ),
    )(embedding, eid, w1_cat, b1_cat, w2_aug)
    return out


# speedup vs baseline: 1.0165x; 1.0165x over previous
import functools

import jax
import jax.numpy as jnp
import numpy as np
from jax import lax
from jax.experimental import pallas as pl
from jax.experimental.pallas import tpu as pltpu
from jax.experimental.pallas import tpu_sc as plsc

N = 32768
D = 768
H = 64
O = 16
E = 8
PER = N // E

TILE = 4096
SC_NS = 16
SC_CHUNK = N // SC_NS
SC_K = SC_CHUNK // 128


def _sc_route_eid(idx3, eid3):
    mesh = plsc.VectorSubcoreMesh(
        core_axis_name="core", subcore_axis_name="subcore", num_cores=1)

    @functools.partial(
        pl.kernel,
        out_type=jax.ShapeDtypeStruct((N,), jnp.int32),
        mesh=mesh,
        scratch_types=[
            pltpu.VMEM((SC_K, 128), jnp.int32),
            pltpu.VMEM((SC_K, 128), jnp.int32),
            pltpu.VMEM((SC_CHUNK,), jnp.int32),
            pltpu.VMEM_SHARED((N,), jnp.int32),
            pltpu.SemaphoreType.DMA,
        ],
    )
    def sc_kernel(idx_hbm, eid_hbm, out_hbm, idx_v, val_v, stage_v, shared, sem):
        sid = lax.axis_index("subcore")
        base = sid * SC_CHUNK
        c_idx = pltpu.async_copy(idx_hbm.at[sid], idx_v, sem)
        c_val = pltpu.async_copy(eid_hbm.at[sid], val_v, sem)
        c_idx.wait()
        c_val.wait()
        scatters = [
            pltpu.async_copy(val_v.at[j], shared.at[idx_v.at[j]], sem)
            for j in range(SC_K)
        ]
        for c in scatters:
            c.wait()
        plsc.subcore_barrier()
        pltpu.sync_copy(shared.at[pl.ds(base, SC_CHUNK)], stage_v)
        pltpu.sync_copy(stage_v, out_hbm.at[pl.ds(base, SC_CHUNK)])

    return sc_kernel(idx3, eid3)


def _mlp_kernel(emb_ref, eid_ref, w1_ref, b1_ref, w2_ref, out_ref):
    x = emb_ref[...].astype(jnp.bfloat16)
    h = jax.lax.dot_general(
        w1_ref[...], x, (((0,), (1,)), ((), ())),
        preferred_element_type=jnp.float32,
    ) + b1_ref[...]
    h = h * jax.nn.sigmoid(h)
    eid = eid_ref[...]
    slot = jax.lax.broadcasted_iota(jnp.int32, (E * H, 1), 0) // H
    hm = jnp.where(eid == slot, h, 0.0).astype(jnp.bfloat16)
    oh8 = jnp.where(
        eid == jax.lax.broadcasted_iota(jnp.int32, (E, 1), 0),
        jnp.float32(1.0), jnp.float32(0.0)).astype(jnp.bfloat16)
    hm_aug = jnp.concatenate([hm, oh8], axis=0)
    out_ref[...] = jax.lax.dot_general(
        hm_aug, w2_ref[...], (((0,), (0,)), ((), ())),
        preferred_element_type=jnp.float32,
    )


def kernel(species, embedding, idx_0, idx_1, idx_2, idx_3, idx_4, idx_5,
           idx_6, idx_7, W1, b1, W2, b2):
    idx_cat = jnp.concatenate(
        [idx_0, idx_1, idx_2, idx_3, idx_4, idx_5, idx_6, idx_7])
    idx3 = idx_cat.astype(jnp.int32).reshape(SC_NS, SC_K, 128)
    eid3 = jnp.broadcast_to(
        jnp.arange(E, dtype=jnp.int32)[:, None], (E, PER)
    ).reshape(SC_NS, SC_K, 128)

    eid = _sc_route_eid(idx3, eid3).reshape(1, N)

    w1_cat = jnp.transpose(W1, (1, 0, 2)).reshape(D, E * H).astype(jnp.bfloat16)
    b1_cat = b1.reshape(E * H, 1)
    w2_aug = jnp.concatenate(
        [W2.reshape(E * H, O), b2], axis=0).astype(jnp.bfloat16)

    out = pl.pallas_call(
        _mlp_kernel,
        grid=(N // TILE,),
        in_specs=[
            pl.BlockSpec((TILE, D), lambda i: (i, 0)),
            pl.BlockSpec((1, TILE), lambda i: (0, i)),
            pl.BlockSpec((D, E * H), lambda i: (0, 0)),
            pl.BlockSpec((E * H, 1), lambda i: (0, 0)),
            pl.BlockSpec((E * H + E, O), lambda i: (0, 0)),
        ],
        out_specs=pl.BlockSpec((TILE, O), lambda i: (i, 0)),
        out_shape=jax.ShapeDtypeStruct((N, O), jnp.float32),
        compiler_params=pltpu.CompilerParams(
            dimension_semantics=("arbitrary",),
        ),
    )(embedding, eid, w1_cat, b1_cat, w2_aug)
    return out
